# baseline (speedup 1.0000x reference)
import jax
import jax.numpy as jnp
from jax.experimental import pallas as pl
from jax.experimental.pallas import tpu as pltpu

N, D, S = 16384, 256, 512
GROUPS = 2
SH = S // GROUPS
BN0 = 2048
NB0 = N // BN0
BN1 = 8192
NB1G = N // BN1
RPB = BN1 // BN0
SPG = NB0 + NB1G

C2 = -1000.0 * 1.4426950408889634


def _kernel(x_ref, s_ref, out_ref, e_scr, mrun_scr, m_scr, l_scr, ssq_scr,
            sneg_scr):
    k = pl.program_id(0)
    c = k // SPG
    t = k % SPG

    @pl.when(t < NB0)
    def _compute():
        @pl.when(k == 0)
        def _precompute():
            s = s_ref[...]
            s_sq = jnp.sum(s * s, axis=1)[None, :]
            for cc in range(GROUPS):
                ssq_scr[pl.ds(cc * 8, 8), :] = jnp.broadcast_to(
                    s_sq[:, cc * SH:(cc + 1) * SH], (8, SH))
            sneg_scr[...] = -2.0 * s

        x = x_ref[...]
        x_sq = jnp.sum(x * x, axis=1, keepdims=True)
        s_sq = ssq_scr[pl.ds(c * 8, 8), :][0:1, :]
        g2 = jax.lax.dot_general(x, sneg_scr[pl.ds(c * SH, SH), :],
                                 (((1,), (1,)), ((), ())),
                                 preferred_element_type=jnp.float32)
        d2 = jnp.maximum((x_sq + g2) + s_sq, 1e-30)
        score = C2 * (d2 * jax.lax.rsqrt(d2))

        tile_max = jnp.max(score, axis=0, keepdims=True)

        @pl.when(t == 0)
        def _init():
            e = jnp.exp2(score - tile_max)
            e_scr[pl.ds(0, BN0), :] = e
            tile_sum = jnp.sum(e, axis=0, keepdims=True)
            m_scr[...] = jnp.broadcast_to(tile_max, (8, SH))
            l_scr[...] = jnp.broadcast_to(tile_sum, (8, SH))
            mrun_scr[pl.ds(0, 8), :] = jnp.broadcast_to(tile_max, (8, SH))

        @pl.when(t > 0)
        def _update():
            m_old = m_scr[...][0:1, :]
            l_old = l_scr[...][0:1, :]
            m_new = jnp.maximum(m_old, tile_max)
            e = jnp.exp2(score - m_new)
            e_scr[pl.ds(t * BN0, BN0), :] = e
            tile_sum = jnp.sum(e, axis=0, keepdims=True)
            l_new = l_old * jnp.exp2(m_old - m_new) + tile_sum
            m_scr[...] = jnp.broadcast_to(m_new, (8, SH))
            l_scr[...] = jnp.broadcast_to(l_new, (8, SH))
            mrun_scr[pl.ds(t * 8, 8), :] = jnp.broadcast_to(m_new, (8, SH))

    @pl.when(t >= NB0)
    def _normalize():
        j = t - NB0
        m = m_scr[...][0:1, :]
        l = l_scr[...][0:1, :]
        for r in range(RPB):
            kb_off = (j * RPB + r) * 8
            m_k = mrun_scr[pl.ds(kb_off, 8), :][0:1, :]
            cvec = jnp.exp2(m_k - m) / l
            out_ref[pl.ds(r * BN0, BN0), :] = (
                e_scr[pl.ds((j * RPB + r) * BN0, BN0), :] * cvec)


def kernel(inputs, samples):
    return pl.pallas_call(
        _kernel,
        grid=(GROUPS * SPG,),
        in_specs=[
            pl.BlockSpec((BN0, D),
                         lambda k: (jnp.minimum(k % SPG, NB0 - 1), 0)),
            pl.BlockSpec((S, D), lambda k: (0, 0)),
        ],
        out_specs=pl.BlockSpec(
            (BN1, SH),
            lambda k: (jnp.maximum(k % SPG - NB0, 0), k // SPG)),
        out_shape=jax.ShapeDtypeStruct((N, S), jnp.float32),
        scratch_shapes=[
            pltpu.VMEM((N, SH), jnp.float32),
            pltpu.VMEM((NB0 * 8, SH), jnp.float32),
            pltpu.VMEM((8, SH), jnp.float32),
            pltpu.VMEM((8, SH), jnp.float32),
            pltpu.VMEM((GROUPS * 8, SH), jnp.float32),
            pltpu.VMEM((S, D), jnp.float32),
        ],
    )(inputs, samples)


# speedup vs baseline: 1.2798x; 1.2798x over previous
import jax
import jax.numpy as jnp
from jax.experimental import pallas as pl
from jax.experimental.pallas import tpu as pltpu

N, D, S = 16384, 256, 512
BN0 = 4096
NB0 = N // BN0
BN1 = 2048
NB1 = N // BN1
RPB = BN1 // BN0


C2 = -1000.0 * 1.4426950408889634


def _kernel(x_ref, s_ref, out_ref, e_scr, mrun_scr, m_scr, l_scr, ssq_scr,
            sneg_scr):
    k = pl.program_id(0)

    @pl.when(k < NB0)
    def _compute():
        @pl.when(k == 0)
        def _precompute():
            s = s_ref[...]
            s_sq = jnp.sum(s * s, axis=1)[None, :]
            ssq_scr[...] = jnp.broadcast_to(s_sq, (8, S))
            sneg_scr[...] = -2.0 * s

        x = x_ref[...]
        x_sq = jnp.sum(x * x, axis=1, keepdims=True)
        s_sq = ssq_scr[...][0:1, :]
        g2 = jax.lax.dot_general(x, sneg_scr[...], (((1,), (1,)), ((), ())),
                                 preferred_element_type=jnp.float32)
        d2 = jnp.maximum((x_sq + g2) + s_sq, 1e-30)
        score = C2 * (d2 * jax.lax.rsqrt(d2))

        tile_max = jnp.max(score, axis=0, keepdims=True)

        @pl.when(k == 0)
        def _init():
            e = jnp.exp2(score - tile_max)
            e_scr[pl.ds(0, BN0), :] = e
            tile_sum = jnp.sum(e, axis=0, keepdims=True)
            m_scr[...] = jnp.broadcast_to(tile_max, (8, S))
            l_scr[...] = jnp.broadcast_to(tile_sum, (8, S))
            mrun_scr[pl.ds(0, 8), :] = jnp.broadcast_to(tile_max, (8, S))

        @pl.when(k > 0)
        def _update():
            m_old = m_scr[...][0:1, :]
            l_old = l_scr[...][0:1, :]
            m_new = jnp.maximum(m_old, tile_max)
            e = jnp.exp2(score - m_new)
            e_scr[pl.ds(k * BN0, BN0), :] = e
            tile_sum = jnp.sum(e, axis=0, keepdims=True)
            l_new = l_old * jnp.exp2(m_old - m_new) + tile_sum
            m_scr[...] = jnp.broadcast_to(m_new, (8, S))
            l_scr[...] = jnp.broadcast_to(l_new, (8, S))
            mrun_scr[pl.ds(k * 8, 8), :] = jnp.broadcast_to(m_new, (8, S))

    @pl.when(k >= NB0)
    def _normalize():
        j = k - NB0
        m = m_scr[...][0:1, :]
        l = l_scr[...][0:1, :]
        for r in range(RPB):
            kb_off = (j * RPB + r) * 8
            m_k = mrun_scr[pl.ds(kb_off, 8), :][0:1, :]
            c = jnp.exp2(m_k - m) / l
            out_ref[pl.ds(r * BN0, BN0), :] = (
                e_scr[pl.ds((j * RPB + r) * BN0, BN0), :] * c)


def kernel(inputs, samples):
    return pl.pallas_call(
        _kernel,
        grid=(NB0 + NB1,),
        in_specs=[
            pl.BlockSpec((BN0, D), lambda k: (jnp.minimum(k, NB0 - 1), 0)),
            pl.BlockSpec((S, D), lambda k: (0, 0)),
        ],
        out_specs=pl.BlockSpec((BN1, S), lambda k: (jnp.maximum(k - NB0, 0), 0)),
        out_shape=jax.ShapeDtypeStruct((N, S), jnp.float32),
        scratch_shapes=[
            pltpu.VMEM((N, S), jnp.float32),
            pltpu.VMEM((NB0 * 8, S), jnp.float32),
            pltpu.VMEM((8, S), jnp.float32),
            pltpu.VMEM((8, S), jnp.float32),
            pltpu.VMEM((8, S), jnp.float32),
            pltpu.VMEM((S, D), jnp.float32),
        ],
    )(inputs, samples)
